# baseline (speedup 1.0000x reference)
import functools

import jax
import jax.numpy as jnp
from jax import lax
from jax.experimental import pallas as pl
from jax.experimental.pallas import tpu as pltpu
from jax.experimental.pallas import tpu_sc as plsc

DIM = 32
LANES = 16
NUM_CORES = 2
NUM_SUBCORES = 16
NUM_WORKERS = NUM_CORES * NUM_SUBCORES


def _make_kernel(batch):
    b_per_w = batch // NUM_WORKERS
    n_groups = b_per_w // LANES
    mesh = plsc.VectorSubcoreMesh(
        core_axis_name="c",
        subcore_axis_name="s",
        num_cores=NUM_CORES,
        num_subcores=NUM_SUBCORES,
    )

    @functools.partial(
        pl.kernel,
        out_type=jax.ShapeDtypeStruct((batch,), jnp.float32),
        mesh=mesh,
        scratch_types=[
            pltpu.VMEM((b_per_w,), jnp.int32),
            pltpu.VMEM((b_per_w,), jnp.int32),
            pltpu.VMEM((DIM, b_per_w), jnp.float32),
            pltpu.VMEM((DIM, b_per_w), jnp.float32),
            pltpu.VMEM((b_per_w,), jnp.float32),
            pltpu.SemaphoreType.DMA,
            pltpu.SemaphoreType.DMA,
        ],
        compiler_params=pltpu.CompilerParams(
            needs_layout_passes=False, use_tc_tiling_on_sc=False),
    )
    def scores_kernel(ut_hbm, it_hbm, uidx_hbm, iidx_hbm, out_hbm,
                      uidx_v, iidx_v, ucols_v, icols_v, scores_v,
                      sem_u, sem_v):
        wid = lax.axis_index("s") * NUM_CORES + lax.axis_index("c")
        base = wid * b_per_w
        pltpu.sync_copy(uidx_hbm.at[pl.ds(base, b_per_w)], uidx_v)
        pltpu.sync_copy(iidx_hbm.at[pl.ds(base, b_per_w)], iidx_v)
        for d in range(DIM):
            pltpu.async_copy(ut_hbm.at[d].at[uidx_v], ucols_v.at[d], sem_u)
            pltpu.async_copy(it_hbm.at[d].at[iidx_v], icols_v.at[d], sem_v)
        for d in range(DIM):
            pltpu.make_async_copy(ut_hbm.at[d].at[uidx_v], ucols_v.at[d],
                                  sem_u).wait()
            pltpu.make_async_copy(it_hbm.at[d].at[iidx_v], icols_v.at[d],
                                  sem_v).wait()

        def group_body(g, carry):
            acc = jnp.zeros((LANES,), jnp.float32)
            for d in range(DIM):
                acc = acc + (ucols_v[d, pl.ds(g * LANES, LANES)] *
                             icols_v[d, pl.ds(g * LANES, LANES)])
            scores_v[pl.ds(g * LANES, LANES)] = acc
            return carry

        lax.fori_loop(0, n_groups, group_body, 0)
        pltpu.sync_copy(scores_v, out_hbm.at[pl.ds(base, b_per_w)])

    return scores_kernel


@jax.jit
def kernel(user_emb, item_emb, user_idx, item_idx):
    batch = user_idx.shape[0]
    fn = _make_kernel(batch)
    return fn(user_emb.T, item_emb.T,
              user_idx.astype(jnp.int32), item_idx.astype(jnp.int32))


# speedup vs baseline: 4.8855x; 4.8855x over previous
import functools

import jax
import jax.numpy as jnp
from jax import lax
from jax.experimental import pallas as pl
from jax.experimental.pallas import tpu as pltpu
from jax.experimental.pallas import tpu_sc as plsc

DIM = 32
LANES = 16
NUM_CORES = 2
NUM_SUBCORES = 16
NUM_WORKERS = NUM_CORES * NUM_SUBCORES


def _make_kernel(batch):
    b_per_w = batch // NUM_WORKERS
    n_groups = b_per_w // LANES
    mesh = plsc.VectorSubcoreMesh(
        core_axis_name="c",
        subcore_axis_name="s",
        num_cores=NUM_CORES,
        num_subcores=NUM_SUBCORES,
    )

    @functools.partial(
        pl.kernel,
        out_type=jax.ShapeDtypeStruct((batch,), jnp.float32),
        mesh=mesh,
        scratch_types=[
            pltpu.VMEM((b_per_w,), jnp.int32),
            pltpu.VMEM((b_per_w,), jnp.int32),
            pltpu.VMEM((b_per_w, DIM), jnp.bfloat16),
            pltpu.VMEM((b_per_w, DIM), jnp.bfloat16),
            pltpu.VMEM((b_per_w * LANES,), jnp.float32),
            pltpu.VMEM((b_per_w,), jnp.float32),
            pltpu.SemaphoreType.DMA,
            pltpu.SemaphoreType.DMA,
        ],
        compiler_params=pltpu.CompilerParams(
            needs_layout_passes=False, use_tc_tiling_on_sc=False),
    )
    def scores_kernel(user_hbm, item_hbm, uidx_hbm, iidx_hbm, out_hbm,
                      uidx_v, iidx_v, urows_v, vrows_v, half_v, scores_v,
                      sem_u, sem_v):
        wid = lax.axis_index("s") * NUM_CORES + lax.axis_index("c")
        base = wid * b_per_w
        pltpu.sync_copy(uidx_hbm.at[pl.ds(base, b_per_w)], uidx_v)
        pltpu.sync_copy(iidx_hbm.at[pl.ds(base, b_per_w)], iidx_v)
        cp_u = pltpu.async_copy(user_hbm.at[uidx_v], urows_v, sem_u)
        cp_v = pltpu.async_copy(item_hbm.at[iidx_v], vrows_v, sem_v)
        cp_u.wait()
        cp_v.wait()

        def row_body(r, carry):
            u = urows_v[r, pl.ds(0, DIM)]
            v = vrows_v[r, pl.ds(0, DIM)]
            u0, u1 = plsc.unpack(u, format=plsc.PackFormat.INTERLEAVED)
            v0, v1 = plsc.unpack(v, format=plsc.PackFormat.INTERLEAVED)
            half_v[pl.ds(r * LANES, LANES)] = u0 * v0 + u1 * v1
            return carry

        lax.fori_loop(0, b_per_w, row_body, 0)

        lane16 = lax.iota(jnp.int32, LANES) * LANES

        def group_body(g, carry):
            gbase = g * (LANES * LANES) + lane16
            acc = jnp.zeros((LANES,), jnp.float32)
            for l in range(LANES):
                acc = acc + plsc.load_gather(half_v, [gbase + l])
            scores_v[pl.ds(g * LANES, LANES)] = acc
            return carry

        lax.fori_loop(0, n_groups, group_body, 0)
        pltpu.sync_copy(scores_v, out_hbm.at[pl.ds(base, b_per_w)])

    return scores_kernel


@jax.jit
def kernel(user_emb, item_emb, user_idx, item_idx):
    batch = user_idx.shape[0]
    fn = _make_kernel(batch)
    return fn(user_emb.astype(jnp.bfloat16), item_emb.astype(jnp.bfloat16),
              user_idx.astype(jnp.int32), item_idx.astype(jnp.int32))


# speedup vs baseline: 5.7209x; 1.1710x over previous
import functools

import jax
import jax.numpy as jnp
from jax import lax
from jax.experimental import pallas as pl
from jax.experimental.pallas import tpu as pltpu
from jax.experimental.pallas import tpu_sc as plsc

DIM = 32
LANES = 16
NUM_CORES = 2
NUM_SUBCORES = 16
NUM_WORKERS = NUM_CORES * NUM_SUBCORES


def _make_kernel(batch):
    b_per_w = batch // NUM_WORKERS
    n_groups = b_per_w // LANES
    mesh = plsc.VectorSubcoreMesh(
        core_axis_name="c",
        subcore_axis_name="s",
        num_cores=NUM_CORES,
        num_subcores=NUM_SUBCORES,
    )

    @functools.partial(
        pl.kernel,
        out_type=jax.ShapeDtypeStruct((batch,), jnp.float32),
        mesh=mesh,
        scratch_types=[
            pltpu.VMEM((b_per_w,), jnp.int32),
            pltpu.VMEM((b_per_w,), jnp.int32),
            pltpu.VMEM((b_per_w, DIM), jnp.float32),
            pltpu.VMEM((b_per_w, DIM), jnp.float32),
            pltpu.VMEM((b_per_w * LANES,), jnp.float32),
            pltpu.VMEM((b_per_w,), jnp.float32),
            pltpu.SemaphoreType.DMA,
            pltpu.SemaphoreType.DMA,
        ],
        compiler_params=pltpu.CompilerParams(
            needs_layout_passes=False, use_tc_tiling_on_sc=False),
    )
    def scores_kernel(user_hbm, item_hbm, uidx_hbm, iidx_hbm, out_hbm,
                      uidx_v, iidx_v, urows_v, vrows_v, half_v, scores_v,
                      sem_u, sem_v):
        wid = lax.axis_index("s") * NUM_CORES + lax.axis_index("c")
        base = wid * b_per_w
        pltpu.sync_copy(uidx_hbm.at[pl.ds(base, b_per_w)], uidx_v)
        pltpu.sync_copy(iidx_hbm.at[pl.ds(base, b_per_w)], iidx_v)
        cp_u = pltpu.async_copy(user_hbm.at[uidx_v], urows_v, sem_u)
        cp_v = pltpu.async_copy(item_hbm.at[iidx_v], vrows_v, sem_v)
        cp_u.wait()
        cp_v.wait()

        def row_body(r, carry):
            u0 = urows_v[r, pl.ds(0, LANES)]
            u1 = urows_v[r, pl.ds(LANES, LANES)]
            v0 = vrows_v[r, pl.ds(0, LANES)]
            v1 = vrows_v[r, pl.ds(LANES, LANES)]
            half_v[pl.ds(r * LANES, LANES)] = u0 * v0 + u1 * v1
            return carry

        lax.fori_loop(0, b_per_w, row_body, 0)

        lane16 = lax.iota(jnp.int32, LANES) * LANES

        def group_body(g, carry):
            gbase = g * (LANES * LANES) + lane16
            acc = jnp.zeros((LANES,), jnp.float32)
            for l in range(LANES):
                acc = acc + plsc.load_gather(half_v, [gbase + l])
            scores_v[pl.ds(g * LANES, LANES)] = acc
            return carry

        lax.fori_loop(0, n_groups, group_body, 0)
        pltpu.sync_copy(scores_v, out_hbm.at[pl.ds(base, b_per_w)])

    return scores_kernel


@jax.jit
def kernel(user_emb, item_emb, user_idx, item_idx):
    batch = user_idx.shape[0]
    fn = _make_kernel(batch)
    return fn(user_emb, item_emb,
              user_idx.astype(jnp.int32), item_idx.astype(jnp.int32))
